# baseline (speedup 1.0000x reference)
import functools

import jax
import jax.numpy as jnp
from jax.experimental import pallas as pl
from jax.experimental.pallas import tpu as pltpu

N = 128
A = 256
C = 1000
BETA = 1.0
CHUNK = 200
NSTEPS = C // CHUNK

_dot_t = functools.partial(
    jax.lax.dot_general,
    dimension_numbers=(((1,), (1,)), ((), ())),
    preferred_element_type=jnp.float32,
)


def _isda_body(labels_ref, tail_ref, wts_ref, x_ref, fc_ref, kgc_ref,
               loss_ref, y_ref, bp_ref, p_ref, h_ref, wl_ref):
    i = pl.program_id(0)
    labels = labels_ref[...]

    cls_iota = jax.lax.broadcasted_iota(jnp.int32, (N, C), 1)
    onehot = (cls_iota == labels).astype(jnp.float32)

    @pl.when(i == 0)
    def _():
        F = x_ref[...]
        W = fc_ref[...]
        P = _dot_t(onehot, onehot)
        p_ref[...] = P
        cnt = jnp.sum(P, axis=1, keepdims=True)
        mean = jnp.dot(P, F, preferred_element_type=jnp.float32) / cnt
        h_ref[...] = (F - mean) ** 2 / cnt
        wl_ref[...] = jnp.dot(onehot, W,
                              preferred_element_type=jnp.float32)
        y_ref[...] = _dot_t(F, W)

    row_iota = (jax.lax.broadcasted_iota(jnp.int32, (N, CHUNK), 1)
                + i * CHUNK)
    oc = (row_iota == labels).astype(jnp.float32)
    inner = _dot_t(kgc_ref[...], onehot)
    part = jnp.dot(oc, inner, preferred_element_type=jnp.float32)

    @pl.when(i == 0)
    def _():
        bp_ref[...] = part

    @pl.when(i > 0)
    def _():
        bp_ref[...] += part

    @pl.when(i == NSTEPS - 1)
    def _():
        F = x_ref[...]
        W = fc_ref[...]
        tail = tail_ref[...]
        wts = wts_ref[...]
        H = h_ref[...]
        Wl = wl_ref[...]
        y = y_ref[...]

        in_tail = jnp.max((labels == tail).astype(jnp.float32),
                          axis=1, keepdims=True)
        mixer = jnp.where(in_tail > 0, bp_ref[...], p_ref[...])
        U = jnp.dot(mixer, H, preferred_element_type=jnp.float32)

        V = U * Wl
        s = jnp.sum(V * Wl, axis=1, keepdims=True)
        Vw = _dot_t(V, W)
        Uw2 = _dot_t(U, W * W)
        Z = y + BETA * (0.5 * Uw2 - Vw + 0.5 * s)

        m = jnp.max(Z, axis=1, keepdims=True)
        lse = m + jnp.log(jnp.sum(jnp.exp(Z - m), axis=1, keepdims=True))
        z_lab = jnp.sum(Z * onehot, axis=1, keepdims=True)
        w_lab = jnp.sum(wts * onehot, axis=1, keepdims=True)
        nll = lse - z_lab
        loss_ref[...] = (jnp.sum(w_lab * nll, keepdims=True)
                         / jnp.sum(w_lab, keepdims=True))


@jax.jit
def kernel(x, target_x, weights, kg_sigma, index_tail, fc_weight):
    labels = target_x.reshape(N, 1)
    tail = index_tail.reshape(1, -1)
    wts = weights.reshape(1, C)

    loss, y = pl.pallas_call(
        _isda_body,
        grid=(NSTEPS,),
        in_specs=[
            pl.BlockSpec((N, 1), lambda i: (0, 0)),
            pl.BlockSpec(tail.shape, lambda i: (0, 0)),
            pl.BlockSpec((1, C), lambda i: (0, 0)),
            pl.BlockSpec((N, A), lambda i: (0, 0)),
            pl.BlockSpec((C, A), lambda i: (0, 0)),
            pl.BlockSpec((CHUNK, C), lambda i: (i, 0)),
        ],
        out_specs=(
            pl.BlockSpec((1, 1), lambda i: (0, 0)),
            pl.BlockSpec((N, C), lambda i: (0, 0)),
        ),
        scratch_shapes=[
            pltpu.VMEM((N, N), jnp.float32),
            pltpu.VMEM((N, N), jnp.float32),
            pltpu.VMEM((N, A), jnp.float32),
            pltpu.VMEM((N, A), jnp.float32),
        ],
        out_shape=(
            jax.ShapeDtypeStruct((1, 1), jnp.float32),
            jax.ShapeDtypeStruct((N, C), jnp.float32),
        ),
    )(labels, tail, wts, x, fc_weight, kg_sigma)
    return (loss[0, 0], y)


# speedup vs baseline: 1.2492x; 1.2492x over previous
import functools

import jax
import jax.numpy as jnp
from jax.experimental import pallas as pl
from jax.experimental.pallas import tpu as pltpu

N = 128
A = 256
C = 1000
BETA = 1.0
NCOPY = 5
CHUNK = C // NCOPY

_dot_t = functools.partial(
    jax.lax.dot_general,
    dimension_numbers=(((1,), (1,)), ((), ())),
    preferred_element_type=jnp.float32,
)


def _isda_body(labels_ref, tail_ref, wts_ref, x_ref, fc_ref, kg_hbm,
               loss_ref, y_hbm, kgs_ref, y_ref, sems, ysem):
    copies = [
        pltpu.make_async_copy(
            kg_hbm.at[pl.ds(j * CHUNK, CHUNK), :],
            kgs_ref.at[pl.ds(j * CHUNK, CHUNK), :],
            sems.at[j],
        )
        for j in range(NCOPY)
    ]
    for c in copies:
        c.start()

    F = x_ref[...]
    W = fc_ref[...]
    labels = labels_ref[...]
    tail = tail_ref[...]
    wts = wts_ref[...]

    cls_iota = jax.lax.broadcasted_iota(jnp.int32, (N, C), 1)
    onehot = (cls_iota == labels).astype(jnp.float32)

    y = _dot_t(F, W)
    y_ref[...] = y
    ycopy = pltpu.make_async_copy(y_ref, y_hbm, ysem)
    ycopy.start()

    P = _dot_t(onehot, onehot)
    cnt = jnp.sum(P, axis=1, keepdims=True)
    mean = jnp.dot(P, F, preferred_element_type=jnp.float32) / cnt
    H = (F - mean) ** 2 / cnt
    Wl = jnp.dot(onehot, W, preferred_element_type=jnp.float32)
    in_tail = jnp.max((labels == tail).astype(jnp.float32),
                      axis=1, keepdims=True)
    w_lab = jnp.sum(wts * onehot, axis=1, keepdims=True)

    for c in copies:
        c.wait()

    kgl = jnp.dot(onehot, kgs_ref[...],
                  preferred_element_type=jnp.float32)
    Bp = _dot_t(kgl, onehot)
    mixer = jnp.where(in_tail > 0, Bp, P)
    U = jnp.dot(mixer, H, preferred_element_type=jnp.float32)

    V = U * Wl
    s = jnp.sum(V * Wl, axis=1, keepdims=True)
    Vw = _dot_t(V, W)
    Uw2 = _dot_t(U, W * W)
    Z = y + BETA * (0.5 * Uw2 - Vw + 0.5 * s)

    m = jnp.max(Z, axis=1, keepdims=True)
    lse = m + jnp.log(jnp.sum(jnp.exp(Z - m), axis=1, keepdims=True))
    z_lab = jnp.sum(Z * onehot, axis=1, keepdims=True)
    nll = lse - z_lab
    loss_ref[...] = (jnp.sum(w_lab * nll, keepdims=True)
                     / jnp.sum(w_lab, keepdims=True))
    ycopy.wait()


@jax.jit
def kernel(x, target_x, weights, kg_sigma, index_tail, fc_weight):
    labels = target_x.reshape(N, 1)
    tail = index_tail.reshape(1, -1)
    wts = weights.reshape(1, C)

    loss, y = pl.pallas_call(
        _isda_body,
        in_specs=[
            pl.BlockSpec((N, 1), lambda: (0, 0)),
            pl.BlockSpec(tail.shape, lambda: (0, 0)),
            pl.BlockSpec((1, C), lambda: (0, 0)),
            pl.BlockSpec((N, A), lambda: (0, 0)),
            pl.BlockSpec((C, A), lambda: (0, 0)),
            pl.BlockSpec(memory_space=pl.ANY),
        ],
        out_specs=(
            pl.BlockSpec((1, 1), lambda: (0, 0)),
            pl.BlockSpec(memory_space=pl.ANY),
        ),
        scratch_shapes=[
            pltpu.VMEM((C, C), jnp.float32),
            pltpu.VMEM((N, C), jnp.float32),
            pltpu.SemaphoreType.DMA((NCOPY,)),
            pltpu.SemaphoreType.DMA,
        ],
        out_shape=(
            jax.ShapeDtypeStruct((1, 1), jnp.float32),
            jax.ShapeDtypeStruct((N, C), jnp.float32),
        ),
    )(labels, tail, wts, x, fc_weight, kg_sigma)
    return (loss[0, 0], y)


# speedup vs baseline: 1.2600x; 1.0086x over previous
import functools

import jax
import jax.numpy as jnp
from jax.experimental import pallas as pl
from jax.experimental.pallas import tpu as pltpu

N = 128
A = 256
C = 1000
BETA = 1.0
NCOPY = 1
CHUNK = C // NCOPY

_dot_t = functools.partial(
    jax.lax.dot_general,
    dimension_numbers=(((1,), (1,)), ((), ())),
    preferred_element_type=jnp.float32,
)


def _isda_body(labels_ref, tail_ref, wts_ref, x_ref, fc_ref, kg_hbm,
               loss_ref, y_hbm, kgs_ref, y_ref, sems, ysem):
    copies = [
        pltpu.make_async_copy(
            kg_hbm.at[pl.ds(j * CHUNK, CHUNK), :],
            kgs_ref.at[pl.ds(j * CHUNK, CHUNK), :],
            sems.at[j],
        )
        for j in range(NCOPY)
    ]
    for c in copies:
        c.start()

    F = x_ref[...]
    W = fc_ref[...]
    labels = labels_ref[...]
    tail = tail_ref[...]
    wts = wts_ref[...]

    cls_iota = jax.lax.broadcasted_iota(jnp.int32, (N, C), 1)
    onehot = (cls_iota == labels).astype(jnp.float32)

    y = _dot_t(F, W)
    y_ref[...] = y
    ycopy = pltpu.make_async_copy(y_ref, y_hbm, ysem)
    ycopy.start()

    P = _dot_t(onehot, onehot)
    cnt = jnp.sum(P, axis=1, keepdims=True)
    mean = jnp.dot(P, F, preferred_element_type=jnp.float32) / cnt
    H = (F - mean) ** 2 / cnt
    Wl = jnp.dot(onehot, W, preferred_element_type=jnp.float32)
    in_tail = jnp.max((labels == tail).astype(jnp.float32),
                      axis=1, keepdims=True)
    w_lab = jnp.sum(wts * onehot, axis=1, keepdims=True)

    for c in copies:
        c.wait()

    kgl = jnp.dot(onehot, kgs_ref[...],
                  preferred_element_type=jnp.float32)
    Bp = _dot_t(kgl, onehot)
    mixer = jnp.where(in_tail > 0, Bp, P)
    U = jnp.dot(mixer, H, preferred_element_type=jnp.float32)

    V = U * Wl
    s = jnp.sum(V * Wl, axis=1, keepdims=True)
    Vw = _dot_t(V, W)
    Uw2 = _dot_t(U, W * W)
    Z = y + BETA * (0.5 * Uw2 - Vw + 0.5 * s)

    m = jnp.max(Z, axis=1, keepdims=True)
    lse = m + jnp.log(jnp.sum(jnp.exp(Z - m), axis=1, keepdims=True))
    z_lab = jnp.sum(Z * onehot, axis=1, keepdims=True)
    nll = lse - z_lab
    loss_ref[...] = (jnp.sum(w_lab * nll, keepdims=True)
                     / jnp.sum(w_lab, keepdims=True))
    ycopy.wait()


@jax.jit
def kernel(x, target_x, weights, kg_sigma, index_tail, fc_weight):
    labels = target_x.reshape(N, 1)
    tail = index_tail.reshape(1, -1)
    wts = weights.reshape(1, C)

    loss, y = pl.pallas_call(
        _isda_body,
        in_specs=[
            pl.BlockSpec((N, 1), lambda: (0, 0)),
            pl.BlockSpec(tail.shape, lambda: (0, 0)),
            pl.BlockSpec((1, C), lambda: (0, 0)),
            pl.BlockSpec((N, A), lambda: (0, 0)),
            pl.BlockSpec((C, A), lambda: (0, 0)),
            pl.BlockSpec(memory_space=pl.ANY),
        ],
        out_specs=(
            pl.BlockSpec((1, 1), lambda: (0, 0)),
            pl.BlockSpec(memory_space=pl.ANY),
        ),
        scratch_shapes=[
            pltpu.VMEM((C, C), jnp.float32),
            pltpu.VMEM((N, C), jnp.float32),
            pltpu.SemaphoreType.DMA((NCOPY,)),
            pltpu.SemaphoreType.DMA,
        ],
        out_shape=(
            jax.ShapeDtypeStruct((1, 1), jnp.float32),
            jax.ShapeDtypeStruct((N, C), jnp.float32),
        ),
    )(labels, tail, wts, x, fc_weight, kg_sigma)
    return (loss[0, 0], y)


# speedup vs baseline: 1.2841x; 1.0191x over previous
import functools

import jax
import jax.numpy as jnp
from jax.experimental import pallas as pl

N = 128
A = 256
C = 1000
BETA = 1.0


def _isda_body(labels_ref, tail_ref, wts_ref, x_ref, fc_ref,
               kg0, kg1, kg2, kg3, kg4, loss_ref, y_ref):
    F = x_ref[...]
    W = fc_ref[...]
    labels = labels_ref[...]
    tail = tail_ref[...]
    wts = wts_ref[...]

    cls_iota = jax.lax.broadcasted_iota(jnp.int32, (N, C), 1)
    onehot = (cls_iota == labels).astype(jnp.float32)

    dot_t = functools.partial(
        jax.lax.dot_general,
        dimension_numbers=(((1,), (1,)), ((), ())),
        preferred_element_type=jnp.float32,
    )

    P = dot_t(onehot, onehot)
    cnt = jnp.sum(P, axis=1, keepdims=True)
    mean = jnp.dot(P, F, preferred_element_type=jnp.float32) / cnt
    H = (F - mean) ** 2 / cnt

    Bp = jnp.zeros((N, N), jnp.float32)
    for j, kgc in enumerate((kg0, kg1, kg2, kg3, kg4)):
        oc = onehot[:, j * 200:(j + 1) * 200]
        inner = dot_t(kgc[...], onehot)
        Bp = Bp + jnp.dot(oc, inner, preferred_element_type=jnp.float32)

    U_tail = jnp.dot(Bp, H, preferred_element_type=jnp.float32)
    U_base = jnp.dot(P, H, preferred_element_type=jnp.float32)
    in_tail = jnp.max((labels == tail).astype(jnp.float32),
                      axis=1, keepdims=True)
    U = jnp.where(in_tail > 0, U_tail, U_base)

    Wl = jnp.dot(onehot, W, preferred_element_type=jnp.float32)
    V = U * Wl
    s = jnp.sum(V * Wl, axis=1, keepdims=True)

    y = dot_t(F, W)
    Vw = dot_t(V, W)
    Uw2 = dot_t(U, W * W)
    Z = y + BETA * (0.5 * Uw2 - Vw + 0.5 * s)

    m = jnp.max(Z, axis=1, keepdims=True)
    lse = m + jnp.log(jnp.sum(jnp.exp(Z - m), axis=1, keepdims=True))
    z_lab = jnp.sum(Z * onehot, axis=1, keepdims=True)
    w_lab = jnp.sum(wts * onehot, axis=1, keepdims=True)
    nll = lse - z_lab
    loss_ref[...] = (jnp.sum(w_lab * nll, keepdims=True)
                     / jnp.sum(w_lab, keepdims=True))
    y_ref[...] = y


@jax.jit
def kernel(x, target_x, weights, kg_sigma, index_tail, fc_weight):
    labels = target_x.reshape(N, 1)
    tail = index_tail.reshape(1, -1)
    wts = weights.reshape(1, C)
    def _full(shape):
        return pl.BlockSpec(shape, lambda i: tuple(0 for _ in shape))

    loss, y = pl.pallas_call(
        _isda_body,
        grid=(1,),
        in_specs=(
            [_full((N, 1)), _full(tail.shape), _full((1, C)),
             _full((N, A)), _full((C, A))]
            + [pl.BlockSpec((200, C), lambda i, j=j: (j, 0))
               for j in range(5)]
        ),
        out_specs=(_full((1, 1)), _full((N, C))),
        out_shape=(
            jax.ShapeDtypeStruct((1, 1), jnp.float32),
            jax.ShapeDtypeStruct((N, C), jnp.float32),
        ),
    )(labels, tail, wts, x, fc_weight, *([kg_sigma] * 5))
    return (loss[0, 0], y)
